# baseline (speedup 1.0000x reference)
import jax
import jax.numpy as jnp
from jax.experimental import pallas as pl


_SEQ_BLK = 256


def _bcast_body(w_ref, out_ref):
    out_ref[0] = w_ref[...]


def kernel(tokens, W_pos):
    batch, seq_len = tokens.shape
    d_model = W_pos.shape[1]
    n_blk = seq_len // _SEQ_BLK
    out = pl.pallas_call(
        _bcast_body,
        grid=(n_blk, batch),
        in_specs=[
            pl.BlockSpec((_SEQ_BLK, d_model), lambda i, b: (i, 0)),
        ],
        out_specs=pl.BlockSpec((1, _SEQ_BLK, d_model), lambda i, b: (b, i, 0)),
        out_shape=jax.ShapeDtypeStruct((batch, seq_len, d_model), W_pos.dtype),
    )(W_pos[:seq_len])
    return out
